# Initial kernel scaffold; baseline (speedup 1.0000x reference)
import jax
import jax.numpy as jnp
from jax.experimental import pallas as pl


def kernel(expression, table):
    raise NotImplementedError("write your pallas kernel here")


# speedup vs baseline: 2.1066x; 2.1066x over previous
import functools

import jax
import jax.numpy as jnp
from jax import lax
from jax.experimental import pallas as pl
from jax.experimental.pallas import tpu as pltpu
from jax.experimental.pallas import tpu_sc as plsc

D = 64
NC = 2
NS = 16
NW = NC * NS
IDX_MINOR = 128
SUB = 4
CHUNK = SUB * IDX_MINOR


def _emb_grid(n_total):
    per_w = n_total // NW
    n_iter = per_w // CHUNK
    rows_per_w = per_w // IDX_MINOR

    mesh = plsc.VectorSubcoreMesh(core_axis_name="c", subcore_axis_name="s")

    @functools.partial(
        pl.kernel,
        mesh=mesh,
        out_type=jax.ShapeDtypeStruct((n_total, D), jnp.float32),
        scratch_types=[
            pltpu.VMEM((SUB, IDX_MINOR), jnp.int32),
            pltpu.VMEM((CHUNK, D), jnp.float32),
            pltpu.SemaphoreType.DMA,
        ],
        compiler_params=pltpu.CompilerParams(use_tc_tiling_on_sc=False),
    )
    def emb(idx_hbm, table_hbm, out_hbm, idx_v, rows_v, sem):
        wid = lax.axis_index("s") * NC + lax.axis_index("c")
        row_base = wid * rows_per_w
        out_base = wid * (rows_per_w * IDX_MINOR)

        def body(i, _):
            pltpu.sync_copy(idx_hbm.at[pl.ds(row_base + i * SUB, SUB), :], idx_v)
            copies = []
            for j in range(SUB):
                copies.append(
                    pltpu.async_copy(
                        table_hbm.at[idx_v.at[j]],
                        rows_v.at[pl.ds(j * IDX_MINOR, IDX_MINOR), :],
                        sem,
                    )
                )
            for c in copies:
                c.wait()
            pltpu.sync_copy(rows_v, out_hbm.at[pl.ds(out_base + i * CHUNK, CHUNK), :])
            return 0

        lax.fori_loop(0, n_iter, body, 0)

    return emb


def kernel(expression, table):
    b, g = expression.shape
    n_total = b * g
    idx2d = expression.reshape(n_total // IDX_MINOR, IDX_MINOR)
    out = _emb_grid(n_total)(idx2d, table)
    return out.reshape(b, g, D)


# speedup vs baseline: 2.1183x; 1.0056x over previous
import functools

import jax
import jax.numpy as jnp
from jax import lax
from jax.experimental import pallas as pl
from jax.experimental.pallas import tpu as pltpu
from jax.experimental.pallas import tpu_sc as plsc

D = 64
NC = 2
NS = 16
NW = NC * NS
IDX_MINOR = 128
SUB = 4
CHUNK = SUB * IDX_MINOR


def _emb_grid(n_total):
    per_w = n_total // NW
    n_iter = per_w // CHUNK
    n_pairs = n_iter // 2
    rows_per_w = per_w // IDX_MINOR
    assert n_iter % 2 == 0 and n_iter >= 4

    mesh = plsc.VectorSubcoreMesh(core_axis_name="c", subcore_axis_name="s")

    @functools.partial(
        pl.kernel,
        mesh=mesh,
        out_type=jax.ShapeDtypeStruct((n_total, D), jnp.float32),
        scratch_types=[
            pltpu.VMEM((rows_per_w, IDX_MINOR), jnp.int32),
            pltpu.VMEM((CHUNK, D), jnp.float32),
            pltpu.VMEM((CHUNK, D), jnp.float32),
            pltpu.SemaphoreType.DMA,
            pltpu.SemaphoreType.DMA,
            pltpu.SemaphoreType.DMA,
            pltpu.SemaphoreType.DMA,
        ],
        compiler_params=pltpu.CompilerParams(use_tc_tiling_on_sc=False),
    )
    def emb(idx_hbm, table_hbm, out_hbm, idx_all, rows0, rows1, sg0, sg1, so0, so1):
        wid = lax.axis_index("s") * NC + lax.axis_index("c")
        row_base = wid * rows_per_w
        out_base = wid * per_w

        pltpu.sync_copy(idx_hbm.at[pl.ds(row_base, rows_per_w), :], idx_all)

        def fire_gather(c, rows_v, sem):
            for j in range(SUB):
                pltpu.async_copy(
                    table_hbm.at[idx_all.at[c * SUB + j]],
                    rows_v.at[pl.ds(j * IDX_MINOR, IDX_MINOR), :],
                    sem,
                )

        def wait_gather(c, rows_v, sem):
            for j in range(SUB):
                pltpu.make_async_copy(
                    table_hbm.at[idx_all.at[c * SUB + j]],
                    rows_v.at[pl.ds(j * IDX_MINOR, IDX_MINOR), :],
                    sem,
                ).wait()

        def fire_out(c, rows_v, sem):
            pltpu.async_copy(
                rows_v, out_hbm.at[pl.ds(out_base + c * CHUNK, CHUNK), :], sem
            )

        def wait_out(c, rows_v, sem):
            pltpu.make_async_copy(
                rows_v, out_hbm.at[pl.ds(out_base + c * CHUNK, CHUNK), :], sem
            ).wait()

        fire_gather(0, rows0, sg0)
        fire_gather(1, rows1, sg1)
        wait_gather(0, rows0, sg0)
        fire_out(0, rows0, so0)

        def body(g, _):
            c0 = 2 * g
            c1 = c0 + 1
            wait_out(c0 - 2, rows0, so0)
            fire_gather(c0, rows0, sg0)
            wait_gather(c0 - 1, rows1, sg1)
            fire_out(c0 - 1, rows1, so1)
            wait_out(c1 - 2, rows1, so1)
            fire_gather(c1, rows1, sg1)
            wait_gather(c0, rows0, sg0)
            fire_out(c0, rows0, so0)
            return 0

        lax.fori_loop(1, n_pairs, body, 0)

        wait_gather(n_iter - 1, rows1, sg1)
        fire_out(n_iter - 1, rows1, so1)
        wait_out(n_iter - 2, rows0, so0)
        wait_out(n_iter - 1, rows1, so1)

    return emb


def kernel(expression, table):
    b, g = expression.shape
    n_total = b * g
    idx2d = expression.reshape(n_total // IDX_MINOR, IDX_MINOR)
    out = _emb_grid(n_total)(idx2d, table)
    return out.reshape(b, g, D)


# speedup vs baseline: 2.3498x; 1.1093x over previous
import functools

import jax
import jax.numpy as jnp
from jax import lax
from jax.experimental import pallas as pl
from jax.experimental.pallas import tpu as pltpu
from jax.experimental.pallas import tpu_sc as plsc

D = 64
NC = 2
NS = 16
NW = NC * NS
L = 16
BBLK = 128
DT = D // 8


def _emb_grid(n_b, n_g, n_tab):
    assert n_b == NW * BBLK
    assert n_g % 2 == 0

    mesh = plsc.VectorSubcoreMesh(core_axis_name="c", subcore_axis_name="s")

    @functools.partial(
        pl.kernel,
        mesh=mesh,
        out_type=jax.ShapeDtypeStruct((n_g, DT, NW, 8, BBLK), jnp.float32),
        scratch_types=[
            pltpu.VMEM((n_g, BBLK), jnp.int32),
            pltpu.VMEM((n_tab,), jnp.float32),
            pltpu.VMEM((DT, 8, BBLK), jnp.float32),
            pltpu.VMEM((DT, 8, BBLK), jnp.float32),
            pltpu.SemaphoreType.DMA,
            pltpu.SemaphoreType.DMA,
        ],
        compiler_params=pltpu.CompilerParams(
            use_tc_tiling_on_sc=False, needs_layout_passes=False
        ),
    )
    def emb(expr_hbm, tab_hbm, out_hbm, idx_v, tab_v, m0, m1, so0, so1):
        wid = lax.axis_index("s") * NC + lax.axis_index("c")
        b0 = wid * BBLK

        pltpu.sync_copy(expr_hbm.at[:, pl.ds(b0, BBLK)], idx_v)
        pltpu.sync_copy(tab_hbm, tab_v)

        def fill(g, m_ref):
            for j in range(BBLK // L):
                idxv = idx_v[g, pl.ds(j * L, L)]
                base = idxv * D
                for d in range(D):
                    m_ref[d // 8, d % 8, pl.ds(j * L, L)] = plsc.load_gather(
                        tab_v, [base + d]
                    )

        def fire_out(g, m_ref, sem):
            pltpu.async_copy(m_ref, out_hbm.at[g, :, wid, :, :], sem)

        def wait_out(g, m_ref, sem):
            pltpu.make_async_copy(m_ref, out_hbm.at[g, :, wid, :, :], sem).wait()

        fill(0, m0)
        fire_out(0, m0, so0)
        fill(1, m1)
        fire_out(1, m1, so1)

        def body(t, _):
            g0 = 2 * t
            g1 = g0 + 1
            wait_out(g0 - 2, m0, so0)
            fill(g0, m0)
            fire_out(g0, m0, so0)
            wait_out(g1 - 2, m1, so1)
            fill(g1, m1)
            fire_out(g1, m1, so1)
            return 0

        lax.fori_loop(1, n_g // 2, body, 0)

        wait_out(n_g - 2, m0, so0)
        wait_out(n_g - 1, m1, so1)

    return emb


def kernel(expression, table):
    b, g = expression.shape
    v, d = table.shape
    expr_t = expression.T
    tab_flat = table.reshape(v * d)
    out5 = _emb_grid(b, g, v * d)(expr_t, tab_flat)
    out = jnp.transpose(out5, (2, 4, 0, 1, 3)).reshape(b, g, d)
    return out


# speedup vs baseline: 9.3217x; 3.9671x over previous
import functools

import jax
import jax.numpy as jnp
from jax import lax
from jax.experimental import pallas as pl
from jax.experimental.pallas import tpu as pltpu
from jax.experimental.pallas import tpu_sc as plsc

D = 64
NC = 2
NS = 16
NW = NC * NS
L = 16
BBLK = 128
DT = D // 8
TS = D + 1


def _emb_grid(n_b, n_g, n_tab):
    assert n_b == NW * BBLK
    assert n_g % 2 == 0

    mesh = plsc.VectorSubcoreMesh(core_axis_name="c", subcore_axis_name="s")

    @functools.partial(
        pl.kernel,
        mesh=mesh,
        out_type=jax.ShapeDtypeStruct((n_g, DT, NW, 8, BBLK), jnp.float32),
        scratch_types=[
            pltpu.VMEM((n_g, BBLK), jnp.int32),
            pltpu.VMEM((n_tab,), jnp.float32),
            pltpu.VMEM((DT, 8, BBLK), jnp.float32),
            pltpu.VMEM((DT, 8, BBLK), jnp.float32),
            pltpu.SemaphoreType.DMA,
            pltpu.SemaphoreType.DMA,
        ],
        compiler_params=pltpu.CompilerParams(
            use_tc_tiling_on_sc=False, needs_layout_passes=False
        ),
    )
    def emb(expr_hbm, tab_hbm, out_hbm, idx_v, tab_v, m0, m1, so0, so1):
        wid = lax.axis_index("s") * NC + lax.axis_index("c")
        b0 = wid * BBLK

        pltpu.sync_copy(expr_hbm.at[:, pl.ds(b0, BBLK)], idx_v)
        pltpu.sync_copy(tab_hbm, tab_v)

        def fill(g, m_ref):
            for j in range(BBLK // L):
                idxv = idx_v[g, pl.ds(j * L, L)]
                base = idxv * TS
                for d0 in range(0, D, L):
                    vals = [
                        plsc.load_gather(tab_v, [base + d])
                        for d in range(d0, d0 + L)
                    ]
                    for k in range(L):
                        d = d0 + k
                        m_ref[d // 8, d % 8, pl.ds(j * L, L)] = vals[k]

        def fire_out(g, m_ref, sem):
            pltpu.async_copy(m_ref, out_hbm.at[g, :, wid, :, :], sem)

        def wait_out(g, m_ref, sem):
            pltpu.make_async_copy(m_ref, out_hbm.at[g, :, wid, :, :], sem).wait()

        fill(0, m0)
        fire_out(0, m0, so0)
        fill(1, m1)
        fire_out(1, m1, so1)

        def body(t, _):
            g0 = 2 * t
            g1 = g0 + 1
            wait_out(g0 - 2, m0, so0)
            fill(g0, m0)
            fire_out(g0, m0, so0)
            wait_out(g1 - 2, m1, so1)
            fill(g1, m1)
            fire_out(g1, m1, so1)
            return 0

        lax.fori_loop(1, n_g // 2, body, 0)

        wait_out(n_g - 2, m0, so0)
        wait_out(n_g - 1, m1, so1)

    return emb


def kernel(expression, table):
    b, g = expression.shape
    v, d = table.shape
    expr_t = expression.T
    tab_flat = jnp.pad(table, ((0, 0), (0, TS - d))).reshape(v * TS)
    out5 = _emb_grid(b, g, v * TS)(expr_t, tab_flat)
    out = jnp.transpose(out5, (2, 4, 0, 1, 3)).reshape(b, g, d)
    return out
